# Initial kernel scaffold; baseline (speedup 1.0000x reference)
import jax
import jax.numpy as jnp
from jax.experimental import pallas as pl


def kernel(x, edge_index, edge_attr, W1, b1, W2, b2):
    raise NotImplementedError("write your pallas kernel here")


# speedup vs baseline: 12.0266x; 12.0266x over previous
import functools

import jax
import jax.numpy as jnp
from jax import lax
from jax.experimental import pallas as pl
from jax.experimental.pallas import tpu as pltpu
from jax.experimental.pallas import tpu_sc as plsc

N = 10000
NP = 10240
F_IN = 128
D = 64
NCLS = 40
NCLSP = 128

NC = 2
NS = 16
NW = NC * NS
L = 16

C = 128
E_REAL = 320000 + N
CHUNKS = -(-E_REAL // (NW * C))
EPT = CHUNKS * C
EP = EPT * NW

ROWS_PT = NP // NW
SLICE_PT = NP // NS

_mesh = plsc.VectorSubcoreMesh(
    core_axis_name="c", subcore_axis_name="s", num_cores=NC, num_subcores=NS)
_sc_params = pltpu.CompilerParams(
    needs_layout_passes=False, use_tc_tiling_on_sc=False)


def _worker_id():
  return lax.axis_index("s") * NC + lax.axis_index("c")


@functools.partial(
    pl.kernel,
    out_type=jax.ShapeDtypeStruct((NW, NP), jnp.float32),
    mesh=_mesh,
    scratch_types=[
        pltpu.VMEM((C,), jnp.int32),
        pltpu.VMEM((C,), jnp.float32),
        pltpu.VMEM((NP,), jnp.float32),
    ],
    compiler_params=_sc_params,
)
def _deg_kernel(col_hbm, ew_hbm, deg_hbm, colb, ewb, degl):
  wid = _worker_id()
  base = wid * EPT

  def zero_body(i, carry):
    degl[pl.ds(i * L, L)] = jnp.zeros((L,), jnp.float32)
    return carry
  lax.fori_loop(0, NP // L, zero_body, 0)

  def chunk_body(it, carry):
    off = base + it * C
    pltpu.sync_copy(col_hbm.at[pl.ds(off, C)], colb)
    pltpu.sync_copy(ew_hbm.at[pl.ds(off, C)], ewb)
    for g in range(C // L):
      cv = colb[pl.ds(g * L, L)]
      ev = ewb[pl.ds(g * L, L)]
      plsc.addupdate_scatter(degl, [cv], ev)
    return carry
  lax.fori_loop(0, CHUNKS, chunk_body, 0)

  pltpu.sync_copy(degl, deg_hbm.at[wid])


def _make_spmm(compute_norm):
  acc_type = jax.ShapeDtypeStruct((NC, NP, D), jnp.float32)
  if compute_norm:
    out_types = [acc_type, jax.ShapeDtypeStruct((EP,), jnp.float32)]
  else:
    out_types = acc_type
  scratch = [
      pltpu.VMEM((C,), jnp.int32),
      pltpu.VMEM((C,), jnp.int32),
      pltpu.VMEM((C,), jnp.float32),
      pltpu.VMEM((C, D), jnp.float32),
      pltpu.VMEM_SHARED((NP, D), jnp.float32),
      pltpu.SemaphoreType.DMA,
  ]
  if compute_norm:
    scratch.insert(3, pltpu.VMEM((C,), jnp.float32))
    scratch.insert(0, pltpu.VMEM((NP,), jnp.float32))

  def body(*refs):
    if compute_norm:
      (row_hbm, col_hbm, ew_hbm, dis_hbm, x_hbm, out_hbm, norm_hbm,
       dis_l, idxr, idxc, nb, ewb, rows, acc, sem) = refs
    else:
      (row_hbm, col_hbm, norm_in_hbm, x_hbm, out_hbm,
       idxr, idxc, nb, rows, acc, sem) = refs
    cid = lax.axis_index("c")
    sid = lax.axis_index("s")
    wid = sid * NC + cid
    base = wid * EPT

    def zrow(i, carry):
      for q in range(D // L):
        rows[i, pl.ds(q * L, L)] = jnp.zeros((L,), jnp.float32)
      return carry
    lax.fori_loop(0, C, zrow, 0)
    for j in range(SLICE_PT // C):
      pltpu.sync_copy(rows, acc.at[pl.ds(sid * SLICE_PT + j * C, C)])
    if compute_norm:
      pltpu.sync_copy(dis_hbm, dis_l)
    plsc.subcore_barrier()

    def chunk_body(it, carry):
      off = base + it * C
      pltpu.sync_copy(row_hbm.at[pl.ds(off, C)], idxr)
      pltpu.sync_copy(col_hbm.at[pl.ds(off, C)], idxc)
      if compute_norm:
        pltpu.sync_copy(ew_hbm.at[pl.ds(off, C)], ewb)
      else:
        pltpu.sync_copy(norm_in_hbm.at[pl.ds(off, C)], nb)
      pltpu.async_copy(x_hbm.at[idxc], rows, sem).wait()
      if compute_norm:
        for g in range(C // L):
          rv = idxr[pl.ds(g * L, L)]
          cv = idxc[pl.ds(g * L, L)]
          dr = plsc.load_gather(dis_l, [rv])
          dc = plsc.load_gather(dis_l, [cv])
          nb[pl.ds(g * L, L)] = dr * ewb[pl.ds(g * L, L)] * dc
        pltpu.sync_copy(nb, norm_hbm.at[pl.ds(off, C)])
      for e in range(C):
        ns = plsc.load_gather(nb, [jnp.full((L,), e, jnp.int32)])
        for q in range(D // L):
          rows[e, pl.ds(q * L, L)] = rows[e, pl.ds(q * L, L)] * ns
      pltpu.sync_copy(rows, acc.at[idxr], add=True)
      return carry
    lax.fori_loop(0, CHUNKS, chunk_body, 0)

    plsc.subcore_barrier()
    pltpu.sync_copy(acc.at[pl.ds(sid * SLICE_PT, SLICE_PT)],
                    out_hbm.at[cid, pl.ds(sid * SLICE_PT, SLICE_PT)])

  return pl.kernel(body, out_type=out_types, mesh=_mesh,
                   scratch_types=scratch, compiler_params=_sc_params)


_spmm_first = _make_spmm(True)
_spmm_next = _make_spmm(False)


def _dis_body(degp_ref, dis_ref):
  deg = jnp.sum(degp_ref[...], axis=0)
  safe = jnp.maximum(deg, 1e-12)
  dis_ref[...] = jnp.where(deg > 0, lax.rsqrt(safe), 0.0)


def _dis_tc(degp):
  return pl.pallas_call(
      _dis_body,
      out_shape=jax.ShapeDtypeStruct((NP // 128, 128), jnp.float32),
  )(degp)


def _temp_body(x_ref, w1_ref, b1_ref, out_ref):
  h = jnp.dot(x_ref[...], w1_ref[...], preferred_element_type=jnp.float32)
  out_ref[...] = jnp.maximum(h + b1_ref[...], 0.0)


def _temp_tc(xp, W1, b1):
  BR = 512
  return pl.pallas_call(
      _temp_body,
      grid=(NP // BR,),
      in_specs=[
          pl.BlockSpec((BR, F_IN), lambda i: (i, 0)),
          pl.BlockSpec((F_IN, D), lambda i: (0, 0)),
          pl.BlockSpec((1, D), lambda i: (0, 0)),
      ],
      out_specs=pl.BlockSpec((BR, D), lambda i: (i, 0)),
      out_shape=jax.ShapeDtypeStruct((NP, D), jnp.float32),
  )(xp, W1, b1)


def _psum_body(p_ref, out_ref):
  out_ref[...] = p_ref[0] + p_ref[1]


def _psum_tc(partials):
  BR = 1024
  return pl.pallas_call(
      _psum_body,
      grid=(NP // BR,),
      in_specs=[pl.BlockSpec((NC, BR, D), lambda i: (0, i, 0))],
      out_specs=pl.BlockSpec((BR, D), lambda i: (i, 0)),
      out_shape=jax.ShapeDtypeStruct((NP, D), jnp.float32),
  )(partials)


def _final_body(temp_ref, s1_ref, q_ref, w2_ref, b2_ref, out_ref):
  s2 = q_ref[0] + q_ref[1]
  wa = w2_ref[0:D] + w2_ref[D:2 * D]
  wb = w2_ref[2 * D:3 * D] + w2_ref[3 * D:4 * D]
  wc = w2_ref[4 * D:5 * D]
  logits = (
      jnp.dot(temp_ref[...], wa, preferred_element_type=jnp.float32)
      + jnp.dot(s1_ref[...], wb, preferred_element_type=jnp.float32)
      + jnp.dot(s2, wc, preferred_element_type=jnp.float32)
      + b2_ref[...])
  mask = lax.broadcasted_iota(jnp.int32, logits.shape, 1) < NCLS
  logits = jnp.where(mask, logits, -jnp.inf)
  m = jnp.max(logits, axis=1, keepdims=True)
  e = jnp.where(mask, jnp.exp(logits - m), 0.0)
  lse = jnp.log(jnp.sum(e, axis=1, keepdims=True)) + m
  out_ref[...] = logits - lse


def _final_tc(temp, s1, q, W2p, b2p):
  BR = 512
  return pl.pallas_call(
      _final_body,
      grid=(NP // BR,),
      in_specs=[
          pl.BlockSpec((BR, D), lambda i: (i, 0)),
          pl.BlockSpec((BR, D), lambda i: (i, 0)),
          pl.BlockSpec((NC, BR, D), lambda i: (0, i, 0)),
          pl.BlockSpec((5 * D, NCLSP), lambda i: (0, 0)),
          pl.BlockSpec((1, NCLSP), lambda i: (0, 0)),
      ],
      out_specs=pl.BlockSpec((BR, NCLSP), lambda i: (i, 0)),
      out_shape=jax.ShapeDtypeStruct((NP, NCLSP), jnp.float32),
  )(temp, s1, q, W2p, b2p)


def kernel(x, edge_index, edge_attr, W1, b1, W2, b2):
  loop = jnp.arange(N, dtype=jnp.int32)
  row = jnp.concatenate([edge_index[0], loop])
  col = jnp.concatenate([edge_index[1], loop])
  ew = jnp.concatenate([edge_attr, jnp.ones((N,), jnp.float32)])
  pad = EP - E_REAL
  row = jnp.pad(row, (0, pad))
  col = jnp.pad(col, (0, pad))
  ew = jnp.pad(ew, (0, pad))

  xp = jnp.pad(x, ((0, NP - N), (0, 0)))
  b1r = b1.reshape(1, D)
  W2p = jnp.pad(W2, ((0, 0), (0, NCLSP - NCLS)))
  b2p = jnp.pad(b2, (0, NCLSP - NCLS)).reshape(1, NCLSP)

  degp = _deg_kernel(col, ew)
  dis = _dis_tc(degp.reshape(NW, NP // 128, 128)).reshape(NP)
  temp = _temp_tc(xp, W1, b1r)

  p1, norm = _spmm_first(row, col, ew, dis, temp)
  s1 = _psum_tc(p1)
  p2 = _spmm_next(row, col, norm, s1)

  out = _final_tc(temp, s1, p2, W2p, b2p)
  return out[:N, :NCLS]
